# baseline (speedup 1.0000x reference)
import functools

import jax
import jax.numpy as jnp
from jax import lax
from jax.experimental import pallas as pl
from jax.experimental.pallas import tpu as pltpu
from jax.experimental.pallas import tpu_sc as plsc

B = 16384
S = 64
D = 192
NPIECE = 13
TROWS = 2 * NPIECE * S
NC = 2
NS = 16
NW = NC * NS
BPW = B // NW
CB = 1
CR = CB * S
NCHUNK = BPW // CB
RPT = TROWS // NS
BBLK = 128
NBLK = BPW // BBLK
CPB = BBLK // CB
NBUF = 4


def _sc_body(board_hbm, piece_hbm, pos_hbm, side_hbm,
             x_hbm,
             tab_piece, tab_pos, tab_side, fused_c, fused_sh,
             board_v, idx_v, rows_v,
             sem_g0, sem_g1, sem_g2, sem_g3,
             sem_s0, sem_s1, sem_s2, sem_s3, sem_b):
    cid = lax.axis_index("c")
    sid = lax.axis_index("s")
    wid = sid * NC + cid

    pltpu.sync_copy(piece_hbm, tab_piece)
    pltpu.sync_copy(pos_hbm, tab_pos)
    pltpu.sync_copy(side_hbm, tab_side)

    def build_row(rl, carry):
        r = sid * RPT + rl
        v = r // S
        sq = r - v * S
        t = v // NPIECE
        p = v - t * NPIECE
        for j in range(D // 16):
            c = j * 16
            fused_c[rl, pl.ds(c, 16)] = (tab_piece[p, pl.ds(c, 16)]
                                         + tab_pos[sq, pl.ds(c, 16)]
                                         + tab_side[t, pl.ds(c, 16)])
        return carry

    lax.fori_loop(0, RPT, build_row, 0)
    pltpu.sync_copy(fused_c, fused_sh.at[pl.ds(sid * RPT, RPT)])
    plsc.subcore_barrier()

    sems_g = (sem_g0, sem_g1, sem_g2, sem_g3)
    sems_s = (sem_s0, sem_s1, sem_s2, sem_s3)
    out_base = wid * (BPW * S)
    lane_iota = lax.iota(jnp.int32, 16)

    def start_board(blk):
        pltpu.async_copy(
            board_hbm.at[pl.ds(wid * BPW + blk * BBLK, BBLK)],
            board_v.at[blk % 2], sem_b)

    def wait_board(blk):
        pltpu.make_async_copy(
            board_hbm.at[pl.ds(wid * BPW + blk * BBLK, BBLK)],
            board_v.at[blk % 2], sem_b).wait()

    def compute_idx(lc, bpar, par):
        for bl in range(CB):
            b = lc * CB + bl
            for q in range(S // 16):
                sq = q * 16
                bd = board_v[bpar, b, pl.ds(sq, 16)]
                idx_v[par, pl.ds(bl * S + sq, 16)] = bd * S + (lane_iota + sq)

    def start_gather(par):
        pltpu.async_copy(fused_sh.at[idx_v.at[par]], rows_v.at[par],
                         sems_g[par])

    def wait_gather(par):
        pltpu.make_async_copy(fused_sh.at[idx_v.at[par]], rows_v.at[par],
                              sems_g[par]).wait()

    def start_scatter(i, par):
        pltpu.async_copy(rows_v.at[par],
                         x_hbm.at[pl.ds(out_base + i * CR, CR)], sems_s[par])

    def wait_scatter(i, par):
        pltpu.make_async_copy(rows_v.at[par],
                              x_hbm.at[pl.ds(out_base + i * CR, CR)],
                              sems_s[par]).wait()

    start_board(0)
    wait_board(0)
    if NBLK > 1:
        start_board(1)

    for blk in range(NBLK):
        bpar = blk % 2
        i0 = blk * CPB
        if blk > 0:
            wait_board(blk)
            if blk + 1 < NBLK:
                start_board(blk + 1)
            for j in range(NBUF - 1):
                compute_idx(j, bpar, j)
                wait_scatter(i0 + j - NBUF, j)
                start_gather(j)
        else:
            for j in range(NBUF - 1):
                compute_idx(j, 0, j)
                start_gather(j)

        def step(lc, buf):
            i = i0 + lc
            nb = (buf + NBUF - 1) % NBUF

            @pl.when(lc + NBUF - 1 < CPB)
            def _advance():
                compute_idx(lc + NBUF - 1, bpar, nb)

                @pl.when(i >= 1)
                def _reclaim():
                    wait_scatter(i - 1, nb)

                start_gather(nb)

            wait_gather(buf)
            start_scatter(i, buf)

        def chunkn(g, carry):
            for buf in range(NBUF):
                step(g * NBUF + buf, buf)
            return carry

        lax.fori_loop(0, CPB // NBUF, chunkn, 0)

    for j in range(NBUF):
        wait_scatter(NCHUNK - NBUF + j, j)


@jax.jit
def _sc_call(board_adj, piece_embed, position_embed, side_embed):
    run = functools.partial(
        pl.kernel,
        mesh=plsc.VectorSubcoreMesh(core_axis_name="c", subcore_axis_name="s"),
        compiler_params=pltpu.CompilerParams(use_tc_tiling_on_sc=False),
        out_type=[
            jax.ShapeDtypeStruct((B * S, D), jnp.float32),
        ],
        scratch_types=[
            pltpu.VMEM((NPIECE, D), jnp.float32),
            pltpu.VMEM((S, D), jnp.float32),
            pltpu.VMEM((2, D), jnp.float32),
            pltpu.VMEM((RPT, D), jnp.float32),
            pltpu.VMEM_SHARED((TROWS, D), jnp.float32),
            pltpu.VMEM((2, BBLK, S), jnp.int32),
            pltpu.VMEM((NBUF, CR), jnp.int32),
            pltpu.VMEM((NBUF, CR, D), jnp.float32),
        ] + [pltpu.SemaphoreType.DMA] * (2 * NBUF + 1),
    )(_sc_body)
    return run(board_adj, piece_embed, position_embed, side_embed)


def kernel(board, side_to_move, piece_embed, position_embed, side_embed):
    board_adj = (board.astype(jnp.int32)
                 + side_to_move.astype(jnp.int32)[:, None] * NPIECE)
    x_flat, = _sc_call(board_adj, piece_embed, position_embed, side_embed)
    return x_flat.reshape(B, S, D)


# speedup vs baseline: 1.0053x; 1.0053x over previous
import functools

import jax
import jax.numpy as jnp
from jax import lax
from jax.experimental import pallas as pl
from jax.experimental.pallas import tpu as pltpu
from jax.experimental.pallas import tpu_sc as plsc

B = 16384
S = 64
D = 192
NPIECE = 13
TROWS = 2 * NPIECE * S
NC = 2
NS = 16
NW = NC * NS
BPW = B // NW
CB = 2
CR = CB * S
NCHUNK = BPW // CB
RPT = TROWS // NS
BBLK = 128
NBLK = BPW // BBLK
CPB = BBLK // CB


def _sc_body(board_hbm, piece_hbm, pos_hbm, side_hbm,
             x_hbm,
             tab_piece, tab_pos, tab_side, fused_c, fused_sh,
             board_v, idx_v, rows_v,
             sem_g0, sem_g1, sem_s0, sem_s1, sem_b):
    cid = lax.axis_index("c")
    sid = lax.axis_index("s")
    wid = sid * NC + cid

    pltpu.sync_copy(piece_hbm, tab_piece)
    pltpu.sync_copy(pos_hbm, tab_pos)
    pltpu.sync_copy(side_hbm, tab_side)

    def build_row(rl, carry):
        r = sid * RPT + rl
        v = r // S
        sq = r - v * S
        t = v // NPIECE
        p = v - t * NPIECE
        for j in range(D // 16):
            c = j * 16
            fused_c[rl, pl.ds(c, 16)] = (tab_piece[p, pl.ds(c, 16)]
                                         + tab_pos[sq, pl.ds(c, 16)]
                                         + tab_side[t, pl.ds(c, 16)])
        return carry

    lax.fori_loop(0, RPT, build_row, 0)
    pltpu.sync_copy(fused_c, fused_sh.at[pl.ds(sid * RPT, RPT)])
    plsc.subcore_barrier()

    sems_g = (sem_g0, sem_g1)
    sems_s = (sem_s0, sem_s1)
    out_base = wid * (BPW * S)
    lane_iota = lax.iota(jnp.int32, 16)

    def start_board(blk):
        pltpu.async_copy(
            board_hbm.at[pl.ds(wid * BPW + blk * BBLK, BBLK)],
            board_v.at[blk % 2], sem_b)

    def wait_board(blk):
        pltpu.make_async_copy(
            board_hbm.at[pl.ds(wid * BPW + blk * BBLK, BBLK)],
            board_v.at[blk % 2], sem_b).wait()

    def compute_idx(lc, bpar, par):
        for bl in range(CB):
            b = lc * CB + bl
            for q in range(S // 16):
                sq = q * 16
                bd = board_v[bpar, b, pl.ds(sq, 16)]
                idx_v[par, pl.ds(bl * S + sq, 16)] = bd * S + (lane_iota + sq)

    def start_gather(par):
        pltpu.async_copy(fused_sh.at[idx_v.at[par]], rows_v.at[par],
                         sems_g[par])

    def wait_gather(par):
        pltpu.make_async_copy(fused_sh.at[idx_v.at[par]], rows_v.at[par],
                              sems_g[par]).wait()

    def start_scatter(i, par):
        pltpu.async_copy(rows_v.at[par],
                         x_hbm.at[pl.ds(out_base + i * CR, CR)], sems_s[par])

    def wait_scatter(i, par):
        pltpu.make_async_copy(rows_v.at[par],
                              x_hbm.at[pl.ds(out_base + i * CR, CR)],
                              sems_s[par]).wait()

    start_board(0)
    wait_board(0)
    if NBLK > 1:
        start_board(1)

    for blk in range(NBLK):
        bpar = blk % 2
        i0 = blk * CPB
        if blk > 0:
            wait_board(blk)
            if blk + 1 < NBLK:
                start_board(blk + 1)
            compute_idx(0, bpar, 0)
            wait_scatter(i0 - 2, 0)
            start_gather(0)
        else:
            compute_idx(0, 0, 0)
            start_gather(0)

        def step(lc, par):
            i = i0 + lc
            nxt = 1 - par

            @pl.when(lc + 1 < CPB)
            def _advance():
                compute_idx(lc + 1, bpar, nxt)

                @pl.when(i >= 1)
                def _reclaim():
                    wait_scatter(i - 1, nxt)

                start_gather(nxt)

            wait_gather(par)
            start_scatter(i, par)

        def chunk2(g, carry):
            for par in range(2):
                step(g * 2 + par, par)
            return carry

        lax.fori_loop(0, CPB // 2, chunk2, 0)

    wait_scatter(NCHUNK - 2, 0)
    wait_scatter(NCHUNK - 1, 1)


@jax.jit
def _sc_call(board_adj, piece_embed, position_embed, side_embed):
    run = functools.partial(
        pl.kernel,
        mesh=plsc.VectorSubcoreMesh(core_axis_name="c", subcore_axis_name="s"),
        compiler_params=pltpu.CompilerParams(use_tc_tiling_on_sc=False),
        out_type=[
            jax.ShapeDtypeStruct((B * S, D), jnp.float32),
        ],
        scratch_types=[
            pltpu.VMEM((NPIECE, D), jnp.float32),
            pltpu.VMEM((S, D), jnp.float32),
            pltpu.VMEM((2, D), jnp.float32),
            pltpu.VMEM((RPT, D), jnp.float32),
            pltpu.VMEM_SHARED((TROWS, D), jnp.float32),
            pltpu.VMEM((2, BBLK, S), jnp.int32),
            pltpu.VMEM((2, CR), jnp.int32),
            pltpu.VMEM((2, CR, D), jnp.float32),
            pltpu.SemaphoreType.DMA,
            pltpu.SemaphoreType.DMA,
            pltpu.SemaphoreType.DMA,
            pltpu.SemaphoreType.DMA,
            pltpu.SemaphoreType.DMA,
        ],
    )(_sc_body)
    return run(board_adj, piece_embed, position_embed, side_embed)


def kernel(board, side_to_move, piece_embed, position_embed, side_embed):
    board_adj = (board.astype(jnp.int32)
                 + side_to_move.astype(jnp.int32)[:, None] * NPIECE)
    x_flat, = _sc_call(board_adj, piece_embed, position_embed, side_embed)
    return x_flat.reshape(B, S, D)
